# baseline (speedup 1.0000x reference)
import functools

import jax
import jax.numpy as jnp
from jax import lax
from jax.experimental import pallas as pl
from jax.experimental.pallas import tpu as pltpu
from jax.experimental.pallas import tpu_sc as plsc

_GROUP = 16
_NUM_WORKERS = 32
_CHUNK_ROWS = 4
_NIN = 2
_NOUT = 4


def _group_sort_2d(n_rows, n_cols):
    rows_per_worker = n_rows // _NUM_WORKERS
    chunks_per_worker = rows_per_worker // _CHUNK_ROWS
    assert n_rows % (_NUM_WORKERS * _CHUNK_ROWS * _NOUT) == 0
    n_t = chunks_per_worker // _NOUT

    mesh = plsc.VectorSubcoreMesh(core_axis_name="c", subcore_axis_name="s")

    @functools.partial(
        pl.kernel,
        out_type=jax.ShapeDtypeStruct((n_rows, n_cols), jnp.float32),
        mesh=mesh,
        scratch_types=(
            [pltpu.VMEM((_CHUNK_ROWS, n_cols), jnp.float32)] * (_NIN + _NOUT)
            + [pltpu.SemaphoreType.DMA] * (_NIN + _NOUT)
        ),
        compiler_params=pltpu.CompilerParams(needs_layout_passes=False),
    )
    def sc_sort(x_hbm, out_hbm, *bufs_and_sems):
        ins = bufs_and_sems[:_NIN]
        outs = bufs_and_sems[_NIN : _NIN + _NOUT]
        sins = bufs_and_sems[_NIN + _NOUT : 2 * _NIN + _NOUT]
        souts = bufs_and_sems[2 * _NIN + _NOUT :]

        wid = lax.axis_index("s") * 2 + lax.axis_index("c")
        base = wid * rows_per_worker

        def in_copy(g, b):
            return pltpu.make_async_copy(
                x_hbm.at[pl.ds(base + g * _CHUNK_ROWS, _CHUNK_ROWS), :],
                ins[b],
                sins[b],
            )

        def out_copy(g, b):
            return pltpu.make_async_copy(
                outs[b],
                out_hbm.at[pl.ds(base + g * _CHUNK_ROWS, _CHUNK_ROWS), :],
                souts[b],
            )

        for b in range(_NIN):
            in_copy(b, b).start()

        def t_body(t, _):
            for b in range(_NOUT):
                g = _NOUT * t + b
                bi = b % _NIN
                ib, ob = ins[bi], outs[b]

                in_copy(g, bi).wait()

                @pl.when(t > 0)
                def _():
                    out_copy(g, b).wait()

                @plsc.parallel_loop(0, n_cols, step=_GROUP, unroll=2)
                def _(i):
                    for r in range(_CHUNK_ROWS):
                        v = ib[r, pl.ds(i, _GROUP)]
                        sorted_keys, _ = plsc.sort_key_val(v, v)
                        ob[r, pl.ds(i, _GROUP)] = sorted_keys

                out_copy(g, b).start()

                if b < _NIN:
                    in_copy(g + _NIN, bi).start()
                else:

                    @pl.when(t < n_t - 1)
                    def _():
                        in_copy(g + _NIN, bi).start()

            return 0

        lax.fori_loop(0, n_t, t_body, 0)

        for b in range(_NOUT):
            out_copy(_NOUT * (n_t - 1) + b, b).wait()

    return sc_sort


def kernel(x):
    shape = x.shape
    n_cols = shape[-1]
    n_rows = x.size // n_cols
    x2 = x.reshape(n_rows, n_cols)
    out = _group_sort_2d(n_rows, n_cols)(x2)
    return out.reshape(shape)


# speedup vs baseline: 1.0139x; 1.0139x over previous
import functools

import jax
import jax.numpy as jnp
from jax import lax
from jax.experimental import pallas as pl
from jax.experimental.pallas import tpu as pltpu
from jax.experimental.pallas import tpu_sc as plsc

_GROUP = 16
_NUM_WORKERS = 32
_CHUNK_ROWS = 2
_NIN = 4
_NOUT = 8


def _group_sort_2d(n_rows, n_cols):
    rows_per_worker = n_rows // _NUM_WORKERS
    chunks_per_worker = rows_per_worker // _CHUNK_ROWS
    assert n_rows % (_NUM_WORKERS * _CHUNK_ROWS * _NOUT) == 0
    n_t = chunks_per_worker // _NOUT

    mesh = plsc.VectorSubcoreMesh(core_axis_name="c", subcore_axis_name="s")

    @functools.partial(
        pl.kernel,
        out_type=jax.ShapeDtypeStruct((n_rows, n_cols), jnp.float32),
        mesh=mesh,
        scratch_types=(
            [pltpu.VMEM((_CHUNK_ROWS, n_cols), jnp.float32)] * (_NIN + _NOUT)
            + [pltpu.SemaphoreType.DMA] * (_NIN + _NOUT)
        ),
        compiler_params=pltpu.CompilerParams(needs_layout_passes=False),
    )
    def sc_sort(x_hbm, out_hbm, *bufs_and_sems):
        ins = bufs_and_sems[:_NIN]
        outs = bufs_and_sems[_NIN : _NIN + _NOUT]
        sins = bufs_and_sems[_NIN + _NOUT : 2 * _NIN + _NOUT]
        souts = bufs_and_sems[2 * _NIN + _NOUT :]

        wid = lax.axis_index("s") * 2 + lax.axis_index("c")
        base = wid * rows_per_worker

        def in_copy(g, b):
            return pltpu.make_async_copy(
                x_hbm.at[pl.ds(base + g * _CHUNK_ROWS, _CHUNK_ROWS), :],
                ins[b],
                sins[b],
            )

        def out_copy(g, b):
            return pltpu.make_async_copy(
                outs[b],
                out_hbm.at[pl.ds(base + g * _CHUNK_ROWS, _CHUNK_ROWS), :],
                souts[b],
            )

        for b in range(_NIN):
            in_copy(b, b).start()

        def t_body(t, _):
            for b in range(_NOUT):
                g = _NOUT * t + b
                bi = b % _NIN
                ib, ob = ins[bi], outs[b]

                in_copy(g, bi).wait()

                @pl.when(t > 0)
                def _():
                    out_copy(g, b).wait()

                @plsc.parallel_loop(0, n_cols, step=_GROUP, unroll=2)
                def _(i):
                    for r in range(_CHUNK_ROWS):
                        v = ib[r, pl.ds(i, _GROUP)]
                        sorted_keys, _ = plsc.sort_key_val(v, v)
                        ob[r, pl.ds(i, _GROUP)] = sorted_keys

                out_copy(g, b).start()

                if b < _NIN:
                    in_copy(g + _NIN, bi).start()
                else:

                    @pl.when(t < n_t - 1)
                    def _():
                        in_copy(g + _NIN, bi).start()

            return 0

        lax.fori_loop(0, n_t, t_body, 0)

        for b in range(_NOUT):
            out_copy(_NOUT * (n_t - 1) + b, b).wait()

    return sc_sort


def kernel(x):
    shape = x.shape
    n_cols = shape[-1]
    n_rows = x.size // n_cols
    x2 = x.reshape(n_rows, n_cols)
    out = _group_sort_2d(n_rows, n_cols)(x2)
    return out.reshape(shape)
